# baseline (speedup 1.0000x reference)
import functools

import jax
import jax.numpy as jnp
from jax import lax
from jax.experimental import pallas as pl
from jax.experimental.pallas import tpu as pltpu
from jax.experimental.pallas import tpu_sc as plsc

NC = 2
NS = 16
CH = 128
WD = 16


def _round_up(a, m):
    return (a + m - 1) // m * m


def _make_sc_deg(NP, K):
    RT = NP // NS
    ZC = RT // CH
    KH = K // 2
    mesh = plsc.VectorSubcoreMesh(core_axis_name="c", subcore_axis_name="s")

    @functools.partial(
        pl.kernel,
        out_type=jax.ShapeDtypeStruct((NC, NP, WD), jnp.float32),
        mesh=mesh,
        compiler_params=pltpu.CompilerParams(use_tc_tiling_on_sc=False),
        scratch_types=[
            pltpu.VMEM((K, CH), jnp.int32),
            pltpu.VMEM((CH, WD), jnp.float32),
            pltpu.VMEM_SHARED((NP, WD), jnp.float32),
            pltpu.SemaphoreType.DMA,
        ],
    )
    def deg_kernel(dst_hbm, ones_hbm, zcol_hbm, out_hbm, idx_v, ones_v, acc, sem):
        c = lax.axis_index("c")
        s = lax.axis_index("s")
        pltpu.sync_copy(dst_hbm.at[s], idx_v)
        pltpu.sync_copy(ones_hbm, ones_v)
        base = s * RT
        for i in range(ZC):
            pltpu.sync_copy(zcol_hbm, acc.at[pl.ds(base + i * CH, CH)])
        plsc.subcore_barrier()
        W = 8
        for cid, lo, hi in ((0, 0, KH), (1, KH, K)):

            @pl.when(c == cid)
            def _():
                descs = [None] * K
                for j in range(lo, hi):
                    if j - lo >= W:
                        descs[j - W].wait()
                    descs[j] = pltpu.async_copy(
                        ones_v, acc.at[idx_v.at[j]], sem, add=True)
                for j in range(max(lo, hi - W), hi):
                    descs[j].wait()

        plsc.subcore_barrier()
        for i in range(ZC):
            pltpu.sync_copy(
                acc.at[pl.ds(base + i * CH, CH)],
                out_hbm.at[c, pl.ds(base + i * CH, CH)],
            )

    return deg_kernel


def _make_sc_agg(NP, K, DH):
    RT = NP // NS
    ZC = RT // CH
    mesh = plsc.VectorSubcoreMesh(core_axis_name="c", subcore_axis_name="s")

    @functools.partial(
        pl.kernel,
        out_type=jax.ShapeDtypeStruct((NP, NC, DH), jnp.float32),
        mesh=mesh,
        compiler_params=pltpu.CompilerParams(use_tc_tiling_on_sc=False),
        scratch_types=[
            pltpu.VMEM((K, CH), jnp.int32),
            pltpu.VMEM((K, CH), jnp.int32),
            pltpu.VMEM((4, CH, DH), jnp.float32),
            pltpu.VMEM_SHARED((NP, DH), jnp.float32),
            pltpu.SemaphoreType.DMA,
            pltpu.SemaphoreType.DMA,
            pltpu.SemaphoreType.DMA,
            pltpu.SemaphoreType.DMA,
            pltpu.SemaphoreType.DMA,
            pltpu.SemaphoreType.DMA,
            pltpu.SemaphoreType.DMA,
            pltpu.SemaphoreType.DMA,
        ],
    )
    def agg_kernel(table_hbm, src2_hbm, dst_hbm, zrow_hbm, out_hbm,
                   srcv, dstv, rows, acc, g0, g1, g2, g3, s0, s1, s2, s3):
        c = lax.axis_index("c")
        s = lax.axis_index("s")
        pltpu.sync_copy(src2_hbm.at[c, s], srcv)
        pltpu.sync_copy(dst_hbm.at[s], dstv)
        base = s * RT
        for i in range(ZC):
            pltpu.sync_copy(zrow_hbm, acc.at[pl.ds(base + i * CH, CH)])
        plsc.subcore_barrier()

        gsems = [g0, g1, g2, g3]
        ssems = [s0, s1, s2, s3]
        NB = 4
        gd = [None] * K
        sd = [None] * K
        waited = set()
        for j in range(min(NB - 1, K)):
            gd[j] = pltpu.async_copy(
                table_hbm.at[srcv.at[j]], rows.at[j % NB], gsems[j % NB])
        for j in range(K):
            b = j % NB
            if j + NB - 1 < K:
                if j >= 1:
                    sd[j - 1].wait()
                    waited.add(j - 1)
                gd[j + NB - 1] = pltpu.async_copy(
                    table_hbm.at[srcv.at[j + NB - 1]],
                    rows.at[(j + NB - 1) % NB], gsems[(j + NB - 1) % NB])
            gd[j].wait()
            sd[j] = pltpu.async_copy(
                rows.at[b], acc.at[dstv.at[j]], ssems[b], add=True)
        for j in range(K):
            if j not in waited:
                sd[j].wait()
        plsc.subcore_barrier()
        for i in range(ZC):
            pltpu.sync_copy(
                acc.at[pl.ds(base + i * CH, CH)],
                out_hbm.at[pl.ds(base + i * CH, CH), c],
            )

    return agg_kernel


def _tc_pre(x, w, b, degp, BLK):
    N, D = x.shape

    def body(x_ref, w_ref, b_ref, degp_ref, o_ref):
        dinv = lax.rsqrt(1.0 + degp_ref[0][:, 0:1] + degp_ref[1][:, 0:1])
        t = lax.dot_general(x_ref[...], w_ref[...],
                            (((1,), (1,)), ((), ())),
                            preferred_element_type=jnp.float32)
        o_ref[...] = (t + b_ref[...]) * dinv

    return pl.pallas_call(
        body,
        grid=(N // BLK,),
        in_specs=[
            pl.BlockSpec((BLK, D), lambda i: (i, 0)),
            pl.BlockSpec((D, D), lambda i: (0, 0)),
            pl.BlockSpec((1, D), lambda i: (0, 0)),
            pl.BlockSpec((2, BLK, WD), lambda i: (0, i, 0)),
        ],
        out_specs=pl.BlockSpec((BLK, D), lambda i: (i, 0)),
        out_shape=jax.ShapeDtypeStruct((N, D), jnp.float32),
    )(x, w, b.reshape(1, D), degp)


def _layer_finish(agg, tprev, dinv, g, beta, relu):
    conv = (agg + tprev) * dinv
    m = jnp.mean(conv, axis=-1, keepdims=True)
    zc = conv - m
    v = jnp.mean(zc * zc, axis=-1, keepdims=True)
    y = zc * lax.rsqrt(v + 1e-5) * g + beta
    if relu:
        y = jnp.maximum(y, 0.0)
    return y


def _tc_mid(agg, tprev, ident, degp, g, beta, wn, bn, BLK):
    N, D = tprev.shape

    def body(agg_ref, tprev_ref, id_ref, degp_ref, g_ref, beta_ref,
             w_ref, b_ref, h_ref, t_ref):
        dinv = lax.rsqrt(1.0 + degp_ref[0][:, 0:1] + degp_ref[1][:, 0:1])
        y = _layer_finish(agg_ref[...], tprev_ref[...], dinv,
                          g_ref[...], beta_ref[...], relu=True)
        h = y + id_ref[...]
        h_ref[...] = h
        t = lax.dot_general(h, w_ref[...], (((1,), (1,)), ((), ())),
                            preferred_element_type=jnp.float32)
        t_ref[...] = (t + b_ref[...]) * dinv

    return pl.pallas_call(
        body,
        grid=(N // BLK,),
        in_specs=[
            pl.BlockSpec((BLK, D), lambda i: (i, 0)),
            pl.BlockSpec((BLK, D), lambda i: (i, 0)),
            pl.BlockSpec((BLK, D), lambda i: (i, 0)),
            pl.BlockSpec((2, BLK, WD), lambda i: (0, i, 0)),
            pl.BlockSpec((1, D), lambda i: (0, 0)),
            pl.BlockSpec((1, D), lambda i: (0, 0)),
            pl.BlockSpec((D, D), lambda i: (0, 0)),
            pl.BlockSpec((1, D), lambda i: (0, 0)),
        ],
        out_specs=[
            pl.BlockSpec((BLK, D), lambda i: (i, 0)),
            pl.BlockSpec((BLK, D), lambda i: (i, 0)),
        ],
        out_shape=[
            jax.ShapeDtypeStruct((N, D), jnp.float32),
            jax.ShapeDtypeStruct((N, D), jnp.float32),
        ],
    )(agg, tprev, ident, degp, g.reshape(1, D), beta.reshape(1, D),
      wn, bn.reshape(1, D))


def _tc_post(agg, tprev, ident, degp, g, beta, BLK):
    N, D = tprev.shape

    def body(agg_ref, tprev_ref, id_ref, degp_ref, g_ref, beta_ref, o_ref):
        dinv = lax.rsqrt(1.0 + degp_ref[0][:, 0:1] + degp_ref[1][:, 0:1])
        y = _layer_finish(agg_ref[...], tprev_ref[...], dinv,
                          g_ref[...], beta_ref[...], relu=False)
        o_ref[...] = y + id_ref[...]

    return pl.pallas_call(
        body,
        grid=(N // BLK,),
        in_specs=[
            pl.BlockSpec((BLK, D), lambda i: (i, 0)),
            pl.BlockSpec((BLK, D), lambda i: (i, 0)),
            pl.BlockSpec((BLK, D), lambda i: (i, 0)),
            pl.BlockSpec((2, BLK, WD), lambda i: (0, i, 0)),
            pl.BlockSpec((1, D), lambda i: (0, 0)),
            pl.BlockSpec((1, D), lambda i: (0, 0)),
        ],
        out_specs=pl.BlockSpec((BLK, D), lambda i: (i, 0)),
        out_shape=jax.ShapeDtypeStruct((N, D), jnp.float32),
    )(agg, tprev, ident, degp, g.reshape(1, D), beta.reshape(1, D))


def kernel(x, edge_index, W0, b0, g0, beta0, W1, b1, g1, beta1,
           W2, b2, g2, beta2):
    N, D = x.shape
    DH = D // 2
    E = edge_index.shape[1]
    NP = _round_up(N + 1, NS * CH)
    K = -(-E // (NS * CH))
    EP = NS * CH * K
    pad = EP - E

    src = edge_index[0]
    dst = edge_index[1]
    srcp = jnp.concatenate(
        [src, jnp.zeros((pad,), jnp.int32)]).reshape(NS, K, CH)
    src2p = jnp.stack([2 * srcp, 2 * srcp + 1], axis=0)
    dstp = jnp.concatenate(
        [dst, jnp.full((pad,), N, jnp.int32)]).reshape(NS, K, CH)
    zrow = jnp.zeros((CH, DH), jnp.float32)
    zcol = jnp.zeros((CH, WD), jnp.float32)
    ones = jnp.ones((CH, WD), jnp.float32)

    sc_deg = _make_sc_deg(NP, K)
    sc_agg = _make_sc_agg(NP, K, DH)
    BLK = 2000 if N % 2000 == 0 else 16

    def to_sc(t):
        return t.reshape(2 * N, DH)

    def to_tc(a):
        return a.reshape(NP, D)[:N]

    degp = sc_deg(dstp, ones, zcol)
    t0 = _tc_pre(x, W0, b0, degp, BLK)
    a0 = sc_agg(to_sc(t0), src2p, dstp, zrow)
    h1, t1 = _tc_mid(to_tc(a0), t0, x, degp, g0, beta0, W1, b1, BLK)
    a1 = sc_agg(to_sc(t1), src2p, dstp, zrow)
    h2, t2 = _tc_mid(to_tc(a1), t1, h1, degp, g1, beta1, W2, b2, BLK)
    a2 = sc_agg(to_sc(t2), src2p, dstp, zrow)
    return _tc_post(to_tc(a2), t2, h2, degp, g2, beta2, BLK)


# speedup vs baseline: 1.2121x; 1.2121x over previous
import functools

import jax
import jax.numpy as jnp
from jax import lax
from jax.experimental import pallas as pl
from jax.experimental.pallas import tpu as pltpu
from jax.experimental.pallas import tpu_sc as plsc

NC = 2
NS = 16
CH = 128
WD = 16


def _round_up(a, m):
    return (a + m - 1) // m * m


def _make_sc_deg(NP, K):
    RT = NP // NS
    ZC = RT // CH
    KH = K // 2
    mesh = plsc.VectorSubcoreMesh(core_axis_name="c", subcore_axis_name="s")

    @functools.partial(
        pl.kernel,
        out_type=jax.ShapeDtypeStruct((NC, NP, WD), jnp.float32),
        mesh=mesh,
        compiler_params=pltpu.CompilerParams(use_tc_tiling_on_sc=False),
        scratch_types=[
            pltpu.VMEM((K, CH), jnp.int32),
            pltpu.VMEM((CH, WD), jnp.float32),
            pltpu.VMEM_SHARED((NP, WD), jnp.float32),
            pltpu.SemaphoreType.DMA,
        ],
    )
    def deg_kernel(dst_hbm, ones_hbm, zcol_hbm, out_hbm, idx_v, ones_v, acc, sem):
        c = lax.axis_index("c")
        s = lax.axis_index("s")
        pltpu.sync_copy(dst_hbm.at[s], idx_v)
        pltpu.sync_copy(ones_hbm, ones_v)
        base = s * RT
        for i in range(ZC):
            pltpu.sync_copy(zcol_hbm, acc.at[pl.ds(base + i * CH, CH)])
        plsc.subcore_barrier()
        W = 8
        for cid, lo, hi in ((0, 0, KH), (1, KH, K)):

            @pl.when(c == cid)
            def _():
                descs = [None] * K
                for j in range(lo, hi):
                    if j - lo >= W:
                        descs[j - W].wait()
                    descs[j] = pltpu.async_copy(
                        ones_v, acc.at[idx_v.at[j]], sem, add=True)
                for j in range(max(lo, hi - W), hi):
                    descs[j].wait()

        plsc.subcore_barrier()
        for i in range(ZC):
            pltpu.sync_copy(
                acc.at[pl.ds(base + i * CH, CH)],
                out_hbm.at[c, pl.ds(base + i * CH, CH)],
            )

    return deg_kernel


def _make_sc_agg(NP, K, DH):
    RT = NP // NS
    ZC = RT // CH
    mesh = plsc.VectorSubcoreMesh(core_axis_name="c", subcore_axis_name="s")

    @functools.partial(
        pl.kernel,
        out_type=jax.ShapeDtypeStruct((NC, NP, DH), jnp.float32),
        mesh=mesh,
        compiler_params=pltpu.CompilerParams(use_tc_tiling_on_sc=False),
        scratch_types=[
            pltpu.VMEM((K, CH), jnp.int32),
            pltpu.VMEM((K, CH), jnp.int32),
            pltpu.VMEM((6, CH, DH), jnp.float32),
            pltpu.VMEM_SHARED((NP, DH), jnp.float32),
            pltpu.SemaphoreType.DMA,
            pltpu.SemaphoreType.DMA,
            pltpu.SemaphoreType.DMA,
            pltpu.SemaphoreType.DMA,
            pltpu.SemaphoreType.DMA,
            pltpu.SemaphoreType.DMA,
            pltpu.SemaphoreType.DMA,
            pltpu.SemaphoreType.DMA,
        ],
    )
    def agg_kernel(table_hbm, src2_hbm, dst_hbm, zrow_hbm, out_hbm,
                   srcv, dstv, rows, acc, g0, g1, g2, g3, s0, s1, s2, s3):
        c = lax.axis_index("c")
        s = lax.axis_index("s")
        base = s * RT
        pre = [pltpu.async_copy(src2_hbm.at[c, s], srcv, g0),
               pltpu.async_copy(dst_hbm.at[s], dstv, g1)]
        pre += [pltpu.async_copy(zrow_hbm, acc.at[pl.ds(base + i * CH, CH)], g2)
                for i in range(ZC)]
        for p in pre:
            p.wait()
        plsc.subcore_barrier()

        gsems = [g0, g1, g2, g3, g0, g1]
        ssems = [s0, s1, s2, s3, s2, s3]
        NB = 6
        gd = [None] * K
        sd = [None] * K
        waited = set()
        for j in range(min(NB - 1, K)):
            gd[j] = pltpu.async_copy(
                table_hbm.at[srcv.at[j]], rows.at[j % NB], gsems[j % NB])
        for j in range(K):
            b = j % NB
            if j + NB - 1 < K:
                if j >= 1:
                    sd[j - 1].wait()
                    waited.add(j - 1)
                gd[j + NB - 1] = pltpu.async_copy(
                    table_hbm.at[srcv.at[j + NB - 1]],
                    rows.at[(j + NB - 1) % NB], gsems[(j + NB - 1) % NB])
            gd[j].wait()
            sd[j] = pltpu.async_copy(
                rows.at[b], acc.at[dstv.at[j]], ssems[b], add=True)
        for j in range(K):
            if j not in waited:
                sd[j].wait()
        plsc.subcore_barrier()
        wb = [pltpu.async_copy(
                  acc.at[pl.ds(base + i * CH, CH)],
                  out_hbm.at[c, pl.ds(base + i * CH, CH)], g3)
              for i in range(ZC)]
        for p in wb:
            p.wait()

    return agg_kernel


def _tc_pre(x, w, b, degp, BLK):
    N, D = x.shape

    def body(x_ref, w_ref, b_ref, degp_ref, o_ref):
        dinv = lax.rsqrt(1.0 + degp_ref[0][:, 0:1] + degp_ref[1][:, 0:1])
        t = lax.dot_general(x_ref[...], w_ref[...],
                            (((1,), (1,)), ((), ())),
                            preferred_element_type=jnp.float32)
        o_ref[...] = (t + b_ref[...]) * dinv

    return pl.pallas_call(
        body,
        grid=(N // BLK,),
        in_specs=[
            pl.BlockSpec((BLK, D), lambda i: (i, 0)),
            pl.BlockSpec((D, D), lambda i: (0, 0)),
            pl.BlockSpec((1, D), lambda i: (0, 0)),
            pl.BlockSpec((2, BLK, WD), lambda i: (0, i, 0)),
        ],
        out_specs=pl.BlockSpec((BLK, D), lambda i: (i, 0)),
        out_shape=jax.ShapeDtypeStruct((N, D), jnp.float32),
    )(x, w, b.reshape(1, D), degp)


def _unpack_agg(p, BLK, DH):
    halves = []
    for cc in range(2):
        e = p[cc][:, :DH]
        o = p[cc][:, DH:]
        halves.append(jnp.stack([e, o], axis=1).reshape(BLK, DH))
    return jnp.concatenate(halves, axis=-1)


def _layer_finish(agg, tprev, dinv, g, beta, relu):
    BLK, D = tprev.shape
    conv = (_unpack_agg(agg, BLK, D // 2) + tprev) * dinv
    m = jnp.mean(conv, axis=-1, keepdims=True)
    zc = conv - m
    v = jnp.mean(zc * zc, axis=-1, keepdims=True)
    y = zc * lax.rsqrt(v + 1e-5) * g + beta
    if relu:
        y = jnp.maximum(y, 0.0)
    return y


def _tc_mid(agg, tprev, ident, degp, g, beta, wn, bn, BLK):
    N, D = tprev.shape

    def body(agg_ref, tprev_ref, id_ref, degp_ref, g_ref, beta_ref,
             w_ref, b_ref, h_ref, t_ref):
        dinv = lax.rsqrt(1.0 + degp_ref[0][:, 0:1] + degp_ref[1][:, 0:1])
        y = _layer_finish(agg_ref[...], tprev_ref[...], dinv,
                          g_ref[...], beta_ref[...], relu=True)
        h = y + id_ref[...]
        h_ref[...] = h
        t = lax.dot_general(h, w_ref[...], (((1,), (1,)), ((), ())),
                            preferred_element_type=jnp.float32)
        t_ref[...] = (t + b_ref[...]) * dinv

    return pl.pallas_call(
        body,
        grid=(N // BLK,),
        in_specs=[
            pl.BlockSpec((2, BLK // 2, D), lambda i: (0, i, 0)),
            pl.BlockSpec((BLK, D), lambda i: (i, 0)),
            pl.BlockSpec((BLK, D), lambda i: (i, 0)),
            pl.BlockSpec((2, BLK, WD), lambda i: (0, i, 0)),
            pl.BlockSpec((1, D), lambda i: (0, 0)),
            pl.BlockSpec((1, D), lambda i: (0, 0)),
            pl.BlockSpec((D, D), lambda i: (0, 0)),
            pl.BlockSpec((1, D), lambda i: (0, 0)),
        ],
        out_specs=[
            pl.BlockSpec((BLK, D), lambda i: (i, 0)),
            pl.BlockSpec((BLK, D), lambda i: (i, 0)),
        ],
        out_shape=[
            jax.ShapeDtypeStruct((N, D), jnp.float32),
            jax.ShapeDtypeStruct((N, D), jnp.float32),
        ],
    )(agg, tprev, ident, degp, g.reshape(1, D), beta.reshape(1, D),
      wn, bn.reshape(1, D))


def _tc_post(agg, tprev, ident, degp, g, beta, BLK):
    N, D = tprev.shape

    def body(agg_ref, tprev_ref, id_ref, degp_ref, g_ref, beta_ref, o_ref):
        dinv = lax.rsqrt(1.0 + degp_ref[0][:, 0:1] + degp_ref[1][:, 0:1])
        y = _layer_finish(agg_ref[...], tprev_ref[...], dinv,
                          g_ref[...], beta_ref[...], relu=False)
        o_ref[...] = y + id_ref[...]

    return pl.pallas_call(
        body,
        grid=(N // BLK,),
        in_specs=[
            pl.BlockSpec((2, BLK // 2, D), lambda i: (0, i, 0)),
            pl.BlockSpec((BLK, D), lambda i: (i, 0)),
            pl.BlockSpec((BLK, D), lambda i: (i, 0)),
            pl.BlockSpec((2, BLK, WD), lambda i: (0, i, 0)),
            pl.BlockSpec((1, D), lambda i: (0, 0)),
            pl.BlockSpec((1, D), lambda i: (0, 0)),
        ],
        out_specs=pl.BlockSpec((BLK, D), lambda i: (i, 0)),
        out_shape=jax.ShapeDtypeStruct((N, D), jnp.float32),
    )(agg, tprev, ident, degp, g.reshape(1, D), beta.reshape(1, D))


def kernel(x, edge_index, W0, b0, g0, beta0, W1, b1, g1, beta1,
           W2, b2, g2, beta2):
    N, D = x.shape
    DH = D // 2
    E = edge_index.shape[1]
    NP = _round_up(N + 1, NS * CH)
    K = -(-E // (NS * CH))
    EP = NS * CH * K
    pad = EP - E

    src = edge_index[0]
    dst = edge_index[1]
    srcp = jnp.concatenate(
        [src, jnp.zeros((pad,), jnp.int32)]).reshape(NS, K, CH)
    src2p = jnp.stack([2 * srcp, 2 * srcp + 1], axis=0)
    dstp = jnp.concatenate(
        [dst, jnp.full((pad,), N, jnp.int32)]).reshape(NS, K, CH)
    zrow = jnp.zeros((CH, DH), jnp.float32)
    zcol = jnp.zeros((CH, WD), jnp.float32)
    ones = jnp.ones((CH, WD), jnp.float32)

    sc_deg = _make_sc_deg(NP, K)
    sc_agg = _make_sc_agg(NP, K, DH)
    BLK = 2000 if N % 2000 == 0 else 16

    def to_sc(t):
        return t.reshape(2 * N, DH)

    def to_tc(a):
        return a.reshape(NC, NP // 2, D)

    degp = sc_deg(dstp, ones, zcol)
    t0 = _tc_pre(x, W0, b0, degp, BLK)
    a0 = sc_agg(to_sc(t0), src2p, dstp, zrow)
    h1, t1 = _tc_mid(to_tc(a0), t0, x, degp, g0, beta0, W1, b1, BLK)
    a1 = sc_agg(to_sc(t1), src2p, dstp, zrow)
    h2, t2 = _tc_mid(to_tc(a1), t1, h1, degp, g1, beta1, W2, b2, BLK)
    a2 = sc_agg(to_sc(t2), src2p, dstp, zrow)
    return _tc_post(to_tc(a2), t2, h2, degp, g2, beta2, BLK)
